# baseline (speedup 1.0000x reference)
import jax
import jax.numpy as jnp
from jax import lax
from jax.experimental import pallas as pl
from jax.experimental.pallas import tpu as pltpu
from jax.experimental.pallas import tpu_sc as plsc

N_DIM = 2048
ROWS = 4 * 4096
NW = 32
ROWS_PER_W = ROWS // NW
BLK_ROWS = 8
N_BLKS = ROWS_PER_W // BLK_ROWS
CHUNKS = N_DIM // 16


def _permute_body(x_hbm, perm_hbm, out_hbm,
                  perm_v, in_v0, in_v1, in_v2, in_v3, out_v0, out_v1,
                  isem0, isem1, isem2, isem3, osem0, osem1):
    wid = lax.axis_index("c") * 16 + lax.axis_index("s")
    w_row = wid * BLK_ROWS

    def in_copy(b, buf, sem):
        return pltpu.make_async_copy(
            x_hbm.at[pl.ds(w_row + b * (NW * BLK_ROWS), BLK_ROWS)], buf, sem)

    def out_copy(b, buf, sem):
        return pltpu.make_async_copy(
            buf, out_hbm.at[pl.ds(w_row + b * (NW * BLK_ROWS), BLK_ROWS)], sem)

    ins = ((in_v0, isem0), (in_v1, isem1), (in_v2, isem2), (in_v3, isem3))
    outs = ((out_v0, osem0), (out_v1, osem1))

    for b0 in range(4):
        in_copy(b0, ins[b0][0], ins[b0][1]).start()
    pltpu.sync_copy(perm_hbm, perm_v)

    row_ids = [jnp.full((16,), r, jnp.int32) for r in range(BLK_ROWS)]

    def group(g, _):
        for p in range(4):
            ib, isem = ins[p]
            ob, osem = outs[p % 2]
            b = g * 4 + p
            in_copy(b, ib, isem).wait()

            @pl.when(b >= 2)
            def _():
                out_copy(b, ob, osem).wait()

            @plsc.parallel_loop(0, CHUNKS, unroll=8)
            def _(k):
                pv = perm_v[pl.ds(k * 16, 16)]
                for r in range(BLK_ROWS):
                    ob[r, pl.ds(k * 16, 16)] = plsc.load_gather(
                        ib, [row_ids[r], pv])

            out_copy(b, ob, osem).start()

            @pl.when(b + 4 < N_BLKS)
            def _():
                in_copy(b + 4, ib, isem).start()
        return 0
    lax.fori_loop(0, N_BLKS // 4, group, 0)

    out_copy(N_BLKS - 2, out_v0, osem0).wait()
    out_copy(N_BLKS - 1, out_v1, osem1).wait()


@jax.jit
def kernel(x, perm):
    shape = x.shape
    x2 = x.reshape(ROWS, N_DIM)
    perm32 = perm.astype(jnp.int32)
    mesh = plsc.VectorSubcoreMesh(core_axis_name="c", subcore_axis_name="s")
    out = pl.kernel(
        _permute_body,
        out_type=jax.ShapeDtypeStruct((ROWS, N_DIM), x.dtype),
        mesh=mesh,
        scratch_types=[
            pltpu.VMEM((N_DIM,), jnp.int32),
            pltpu.VMEM((BLK_ROWS, N_DIM), jnp.float32),
            pltpu.VMEM((BLK_ROWS, N_DIM), jnp.float32),
            pltpu.VMEM((BLK_ROWS, N_DIM), jnp.float32),
            pltpu.VMEM((BLK_ROWS, N_DIM), jnp.float32),
            pltpu.VMEM((BLK_ROWS, N_DIM), jnp.float32),
            pltpu.VMEM((BLK_ROWS, N_DIM), jnp.float32),
            pltpu.SemaphoreType.DMA,
            pltpu.SemaphoreType.DMA,
            pltpu.SemaphoreType.DMA,
            pltpu.SemaphoreType.DMA,
            pltpu.SemaphoreType.DMA,
            pltpu.SemaphoreType.DMA,
        ],
        compiler_params=pltpu.CompilerParams(needs_layout_passes=False),
    )(x2, perm32)
    return out.reshape(shape)


# speedup vs baseline: 1.0077x; 1.0077x over previous
import jax
import jax.numpy as jnp
from jax import lax
from jax.experimental import pallas as pl
from jax.experimental.pallas import tpu as pltpu
from jax.experimental.pallas import tpu_sc as plsc

N_DIM = 2048
ROWS = 4 * 4096
NW = 32
ROWS_PER_W = ROWS // NW
BLK_ROWS = 8
N_BLKS = ROWS_PER_W // BLK_ROWS
CHUNKS = N_DIM // 16


def _permute_body(x_hbm, perm_hbm, out_hbm,
                  perm_v, in_v0, in_v1, out_v0, out_v1,
                  isem0, isem1, osem0, osem1):
    wid = lax.axis_index("c") * 16 + lax.axis_index("s")
    w_row = wid * BLK_ROWS

    def in_copy(b, buf, sem):
        return pltpu.make_async_copy(
            x_hbm.at[pl.ds(w_row + b * (NW * BLK_ROWS), BLK_ROWS)], buf, sem)

    def out_copy(b, buf, sem):
        return pltpu.make_async_copy(
            buf, out_hbm.at[pl.ds(w_row + b * (NW * BLK_ROWS), BLK_ROWS)], sem)

    bufs = ((in_v0, out_v0, isem0, osem0), (in_v1, out_v1, isem1, osem1))

    in_copy(0, in_v0, isem0).start()
    in_copy(1, in_v1, isem1).start()
    pltpu.sync_copy(perm_hbm, perm_v)

    row_ids = [jnp.full((16,), r, jnp.int32) for r in range(BLK_ROWS)]

    def group(g, _):
        for p, (ib, ob, isem, osem) in enumerate(bufs):
            b = g * 2 + p
            in_copy(b, ib, isem).wait()

            @pl.when(b >= 2)
            def _():
                out_copy(b, ob, osem).wait()

            @plsc.parallel_loop(0, CHUNKS, unroll=8)
            def _(k):
                pv = perm_v[pl.ds(k * 16, 16)]
                for r in range(BLK_ROWS):
                    ob[r, pl.ds(k * 16, 16)] = plsc.load_gather(
                        ib, [row_ids[r], pv])

            out_copy(b, ob, osem).start()

            @pl.when(b + 2 < N_BLKS)
            def _():
                in_copy(b + 2, ib, isem).start()
        return 0
    lax.fori_loop(0, N_BLKS // 2, group, 0)

    out_copy(N_BLKS - 2, out_v0, osem0).wait()
    out_copy(N_BLKS - 1, out_v1, osem1).wait()


@jax.jit
def kernel(x, perm):
    shape = x.shape
    x2 = x.reshape(ROWS, N_DIM)
    perm32 = perm.astype(jnp.int32)
    mesh = plsc.VectorSubcoreMesh(core_axis_name="c", subcore_axis_name="s")
    out = pl.kernel(
        _permute_body,
        out_type=jax.ShapeDtypeStruct((ROWS, N_DIM), x.dtype),
        mesh=mesh,
        scratch_types=[
            pltpu.VMEM((N_DIM,), jnp.int32),
            pltpu.VMEM((BLK_ROWS, N_DIM), jnp.float32),
            pltpu.VMEM((BLK_ROWS, N_DIM), jnp.float32),
            pltpu.VMEM((BLK_ROWS, N_DIM), jnp.float32),
            pltpu.VMEM((BLK_ROWS, N_DIM), jnp.float32),
            pltpu.SemaphoreType.DMA,
            pltpu.SemaphoreType.DMA,
            pltpu.SemaphoreType.DMA,
            pltpu.SemaphoreType.DMA,
        ],
        compiler_params=pltpu.CompilerParams(needs_layout_passes=False),
    )(x2, perm32)
    return out.reshape(shape)
